# baseline (speedup 1.0000x reference)
import functools
import jax
import jax.numpy as jnp
from jax import lax
from jax.experimental import pallas as pl
from jax.experimental.pallas import tpu as pltpu
from jax.experimental.pallas import tpu_sc as plsc

_VOCAB = 6
_NAUX = 10
_H = 768
_R = 64


def kernel(input_ids, aux_features):
    B, S = input_ids.shape
    N = B * S
    info = plsc.get_sparse_core_info()
    NC, NS = info.num_cores, info.num_subcores
    NW = NC * NS
    rows_per_w = N // NW
    n_chunks = rows_per_w // _R

    ids = input_ids.reshape(N).astype(jnp.int32)
    aux = aux_features.reshape(N * _NAUX)

    mesh = plsc.VectorSubcoreMesh(core_axis_name="c", subcore_axis_name="s")

    @functools.partial(
        pl.kernel, mesh=mesh,
        out_type=jax.ShapeDtypeStruct((N, _H), jnp.float32),
        compiler_params=pltpu.CompilerParams(needs_layout_passes=False),
        scratch_types=[
            pltpu.VMEM((_R, _H), jnp.float32),
            pltpu.VMEM((_R,), jnp.int32),
            pltpu.VMEM((_R * _NAUX,), jnp.float32),
        ],
    )
    def sc_k(ids_hbm, aux_hbm, out_hbm, rows_v, ids_v, aux_v):
        wid = lax.axis_index("s") * NC + lax.axis_index("c")
        base_w = wid * rows_per_w

        zeros16 = jnp.zeros((16,), jnp.float32)
        ones16 = jnp.full((16,), 1.0, jnp.float32)

        def zrow(r, carry):
            def zcol(c, carry2):
                rows_v[r, pl.ds(c * 16, 16)] = zeros16
                return carry2
            return lax.fori_loop(0, _H // 16, zcol, carry)
        lax.fori_loop(0, _R, zrow, 0)

        def chunk_body(k, carry):
            base = base_w + k * _R
            pltpu.sync_copy(ids_hbm.at[pl.ds(base, _R)], ids_v)
            pltpu.sync_copy(aux_hbm.at[pl.ds(base * _NAUX, _R * _NAUX)], aux_v)

            def set_aux(g, carry2):
                q = g * 16 + lax.iota(jnp.int32, 16)
                r = q // _NAUX
                c = _VOCAB + (q - r * _NAUX)
                vals = aux_v[pl.ds(g * 16, 16)]
                plsc.store_scatter(rows_v, [r, c], vals)
                return carry2
            lax.fori_loop(0, _R * _NAUX // 16, set_aux, 0)

            def set_ones(g, carry2):
                row16 = g * 16 + lax.iota(jnp.int32, 16)
                idv = ids_v[pl.ds(g * 16, 16)]
                plsc.store_scatter(rows_v, [row16, idv], ones16)
                return carry2
            lax.fori_loop(0, _R // 16, set_ones, 0)

            pltpu.sync_copy(rows_v, out_hbm.at[pl.ds(base, _R), :])

            def set_zeros(g, carry2):
                row16 = g * 16 + lax.iota(jnp.int32, 16)
                idv = ids_v[pl.ds(g * 16, 16)]
                plsc.store_scatter(rows_v, [row16, idv], zeros16)
                return carry2
            lax.fori_loop(0, _R // 16, set_zeros, 0)
            return carry
        lax.fori_loop(0, n_chunks, chunk_body, 0)

    out = sc_k(ids, aux)
    return out.reshape(B, S, _H)
